# Initial kernel scaffold; baseline (speedup 1.0000x reference)
import jax
import jax.numpy as jnp
from jax.experimental import pallas as pl


def kernel(features, edge_index, W_self1, W_neigh1, b1, W_self2, W_neigh2, b2, W_mlp1, b_mlp1, W_mlp2, b_mlp2):
    raise NotImplementedError("write your pallas kernel here")


# speedup vs baseline: 2.5487x; 2.5487x over previous
import functools

import jax
import jax.numpy as jnp
from jax import lax
from jax.experimental import pallas as pl
from jax.experimental.pallas import tpu as pltpu
from jax.experimental.pallas import tpu_sc as plsc

N = 10000
E = 160000
IN_F = 256
HID_F = 512
OUT_F = 256

NROW = 10240
EPAD = 163840
S = 80
RPS = NROW // 16
RB = 1000


def _make_sc_agg(ncc, with_deg):
    P = ncc // 2
    W = 8
    mesh = plsc.VectorSubcoreMesh(core_axis_name="c", subcore_axis_name="s")
    out_type = [jax.ShapeDtypeStruct((ncc, NROW, 128), jnp.float32)]
    scratch = [
        pltpu.VMEM((W, 128), jnp.int32),
        pltpu.VMEM((W, 128), jnp.int32),
        pltpu.VMEM((128, 128), jnp.float32),
        pltpu.VMEM_SHARED((NROW, 128), jnp.float32),
    ]
    if with_deg:
        out_type.append(jax.ShapeDtypeStruct((NROW, 128), jnp.float32))

    def body(*refs):
        if with_deg:
            (x_hbm, si_hbm, di_hbm, z64_hbm, ones_hbm,
             agg_hbm, deg_hbm,
             srcv, dstv, rows, acc_sh) = refs
        else:
            (x_hbm, si_hbm, di_hbm, z64_hbm,
             agg_hbm,
             srcv, dstv, rows, acc_sh) = refs
        c = lax.axis_index("c")
        s = lax.axis_index("s")
        for p in range(P):
            @pl.loop(0, RPS, step=64)
            def _(r):
                pltpu.sync_copy(z64_hbm, acc_sh.at[pl.ds(s * RPS + r, 64)])

            plsc.subcore_barrier()

            @pl.loop(0, S, step=W)
            def _(w):
                pltpu.sync_copy(si_hbm.at[p, c, s, pl.ds(w, W)], srcv)
                pltpu.sync_copy(di_hbm.at[s, pl.ds(w, W)], dstv)

                @pl.loop(0, W)
                def _(j):
                    pltpu.sync_copy(x_hbm.at[srcv.at[j]], rows)
                    pltpu.sync_copy(rows, acc_sh.at[dstv.at[j]], add=True)

            plsc.subcore_barrier()

            pltpu.sync_copy(acc_sh.at[pl.ds(s * RPS, RPS)],
                            agg_hbm.at[2 * p + c, pl.ds(s * RPS, RPS)])

        if with_deg:
            @pl.loop(0, RPS, step=64)
            def _(r):
                pltpu.sync_copy(z64_hbm, acc_sh.at[pl.ds(s * RPS + r, 64)])

            plsc.subcore_barrier()

            @pl.when(c == 0)
            def _():
                pltpu.sync_copy(ones_hbm, rows)

                @pl.loop(0, S, step=W)
                def _(w):
                    pltpu.sync_copy(di_hbm.at[s, pl.ds(w, W)], dstv)

                    @pl.loop(0, W)
                    def _(j):
                        pltpu.sync_copy(rows, acc_sh.at[dstv.at[j]], add=True)

            plsc.subcore_barrier()

            @pl.when(c == 0)
            def _():
                pltpu.sync_copy(acc_sh.at[pl.ds(s * RPS, RPS)],
                                deg_hbm.at[pl.ds(s * RPS, RPS)])

    return pl.kernel(body, out_type=tuple(out_type), mesh=mesh,
                     scratch_types=tuple(scratch))


def _tc_layer1(x, agg, inv, ws, wn, b):
    def body(x_ref, a_ref, i_ref, ws_ref, wn_ref, b_ref, o_ref):
        hn = a_ref[...] * i_ref[...]
        acc = jnp.dot(x_ref[...], ws_ref[...], preferred_element_type=jnp.float32)
        acc = acc + jnp.dot(hn, wn_ref[...], preferred_element_type=jnp.float32)
        o_ref[...] = jnp.maximum(acc + b_ref[...], 0.0)

    return pl.pallas_call(
        body,
        grid=(N // RB,),
        in_specs=[
            pl.BlockSpec((RB, IN_F), lambda i: (i, 0)),
            pl.BlockSpec((RB, IN_F), lambda i: (i, 0)),
            pl.BlockSpec((RB, 1), lambda i: (i, 0)),
            pl.BlockSpec((IN_F, HID_F), lambda i: (0, 0)),
            pl.BlockSpec((IN_F, HID_F), lambda i: (0, 0)),
            pl.BlockSpec((1, HID_F), lambda i: (0, 0)),
        ],
        out_specs=pl.BlockSpec((RB, HID_F), lambda i: (i, 0)),
        out_shape=jax.ShapeDtypeStruct((N, HID_F), jnp.float32),
    )(x, agg, inv, ws, wn, b)


def _tc_layer2_mlp(h1, agg, inv, ws, wn, b, wm1, bm1, wm2, bm2):
    def body(h_ref, a_ref, i_ref, ws_ref, wn_ref, b_ref,
             wm1_ref, bm1_ref, wm2_ref, bm2_ref, o_ref):
        hn = a_ref[...] * i_ref[...]
        acc = jnp.dot(h_ref[...], ws_ref[...], preferred_element_type=jnp.float32)
        acc = acc + jnp.dot(hn, wn_ref[...], preferred_element_type=jnp.float32)
        h2 = jnp.maximum(acc + b_ref[...], 0.0)
        t = jnp.dot(h2, wm1_ref[...], preferred_element_type=jnp.float32)
        t = jnp.maximum(t + bm1_ref[...], 0.0)
        o = jnp.dot(t, wm2_ref[...], preferred_element_type=jnp.float32)
        o_ref[...] = o + bm2_ref[...]

    return pl.pallas_call(
        body,
        grid=(N // RB,),
        in_specs=[
            pl.BlockSpec((RB, HID_F), lambda i: (i, 0)),
            pl.BlockSpec((RB, HID_F), lambda i: (i, 0)),
            pl.BlockSpec((RB, 1), lambda i: (i, 0)),
            pl.BlockSpec((HID_F, HID_F), lambda i: (0, 0)),
            pl.BlockSpec((HID_F, HID_F), lambda i: (0, 0)),
            pl.BlockSpec((1, HID_F), lambda i: (0, 0)),
            pl.BlockSpec((HID_F, HID_F), lambda i: (0, 0)),
            pl.BlockSpec((1, HID_F), lambda i: (0, 0)),
            pl.BlockSpec((HID_F, OUT_F), lambda i: (0, 0)),
            pl.BlockSpec((1, OUT_F), lambda i: (0, 0)),
        ],
        out_specs=pl.BlockSpec((RB, OUT_F), lambda i: (i, 0)),
        out_shape=jax.ShapeDtypeStruct((N, OUT_F), jnp.float32),
    )(h1, agg, inv, ws, wn, b, wm1, bm1, wm2, bm2)


def kernel(features, edge_index, W_self1, W_neigh1, b1, W_self2, W_neigh2, b2,
           W_mlp1, b_mlp1, W_mlp2, b_mlp2):
    src = edge_index[0].astype(jnp.int32)
    dst = edge_index[1].astype(jnp.int32)
    srcp = jnp.pad(src, (0, EPAD - E))
    dstp = jnp.pad(dst, (0, EPAD - E), constant_values=N)
    di = dstp.reshape(16, S, 128)
    si_base = srcp.reshape(16, S, 128)

    z64 = jnp.zeros((64, 128), jnp.float32)
    ones128 = jnp.ones((128, 128), jnp.float32)

    xcat1 = features.reshape(N, 2, 128).transpose(1, 0, 2).reshape(2 * N, 128)
    si1 = jnp.stack([si_base, si_base + N])[None]
    agg1_c, deg_raw = _make_sc_agg(2, True)(xcat1, si1, di, z64, ones128)
    agg1 = agg1_c[:, :N, :].transpose(1, 0, 2).reshape(N, IN_F)
    inv = 1.0 / jnp.maximum(deg_raw[:N, 0:1], 1.0)

    h1 = _tc_layer1(features, agg1, inv, W_self1, W_neigh1,
                    b1.reshape(1, HID_F))

    xcat2 = h1.reshape(N, 4, 128).transpose(1, 0, 2).reshape(4 * N, 128)
    si2 = jnp.stack([
        jnp.stack([si_base, si_base + N]),
        jnp.stack([si_base + 2 * N, si_base + 3 * N]),
    ])
    (agg2_c,) = _make_sc_agg(4, False)(xcat2, si2, di, z64)
    agg2 = agg2_c[:, :N, :].transpose(1, 0, 2).reshape(N, HID_F)

    out = _tc_layer2_mlp(h1, agg2, inv, W_self2, W_neigh2,
                         b2.reshape(1, HID_F), W_mlp1, b_mlp1.reshape(1, HID_F),
                         W_mlp2, b_mlp2.reshape(1, OUT_F))
    return out


# speedup vs baseline: 2.9625x; 1.1624x over previous
import functools

import jax
import jax.numpy as jnp
from jax import lax
from jax.experimental import pallas as pl
from jax.experimental.pallas import tpu as pltpu
from jax.experimental.pallas import tpu_sc as plsc

N = 10000
E = 160000
IN_F = 256
HID_F = 512
OUT_F = 256

NROW = 10240
EPAD = 163840
S = 80
RPS = NROW // 16
RB = 1000


def _make_sc_agg(ncc, with_deg):
    P = ncc // 2
    W = 8
    mesh = plsc.VectorSubcoreMesh(core_axis_name="c", subcore_axis_name="s")
    out_type = [jax.ShapeDtypeStruct((ncc, NROW, 128), jnp.float32)]
    scratch = [
        pltpu.VMEM((W, 128), jnp.int32),
        pltpu.VMEM((W, 128), jnp.int32),
        pltpu.VMEM((128, 128), jnp.float32),
        pltpu.VMEM((128, 128), jnp.float32),
        pltpu.VMEM_SHARED((NROW, 128), jnp.float32),
        pltpu.SemaphoreType.DMA,
        pltpu.SemaphoreType.DMA,
    ]
    if with_deg:
        out_type.append(jax.ShapeDtypeStruct((2, NROW, 128), jnp.float32))

    def body(*refs):
        if with_deg:
            (x_hbm, si_hbm, di_hbm, z64_hbm, ones_hbm,
             agg_hbm, deg_hbm,
             srcv, dstv, rows0, rows1, acc_sh, sem0, sem1) = refs
        else:
            (x_hbm, si_hbm, di_hbm, z64_hbm,
             agg_hbm,
             srcv, dstv, rows0, rows1, acc_sh, sem0, sem1) = refs
        c = lax.axis_index("c")
        s = lax.axis_index("s")

        def zero_acc():
            descs = [
                pltpu.async_copy(z64_hbm,
                                 acc_sh.at[pl.ds(s * RPS + r, 64)], sem0)
                for r in range(0, RPS, 64)
            ]
            for d in descs:
                d.wait()

        for p in range(P):
            zero_acc()
            plsc.subcore_barrier()

            @pl.loop(0, S, step=W)
            def _(w):
                pltpu.sync_copy(si_hbm.at[p, c, s, pl.ds(w, W)], srcv)
                pltpu.sync_copy(di_hbm.at[s, pl.ds(w, W)], dstv)
                bufs = (rows0, rows1)
                sems = (sem0, sem1)
                d = pltpu.async_copy(x_hbm.at[srcv.at[0]], rows0, sem0)
                for j in range(W):
                    if j + 1 < W:
                        dn = pltpu.async_copy(x_hbm.at[srcv.at[j + 1]],
                                              bufs[(j + 1) % 2],
                                              sems[(j + 1) % 2])
                    d.wait()
                    pltpu.sync_copy(bufs[j % 2], acc_sh.at[dstv.at[j]],
                                    add=True)
                    if j + 1 < W:
                        d = dn

            plsc.subcore_barrier()

            pltpu.sync_copy(acc_sh.at[pl.ds(s * RPS, RPS)],
                            agg_hbm.at[2 * p + c, pl.ds(s * RPS, RPS)])

        if with_deg:
            zero_acc()
            pltpu.sync_copy(ones_hbm, rows0)
            plsc.subcore_barrier()

            H = S // 2

            @pl.loop(0, H, step=W)
            def _(w0):
                w = w0 + c * H
                pltpu.sync_copy(di_hbm.at[s, pl.ds(w, W)], dstv)
                descs = [
                    pltpu.async_copy(rows0, acc_sh.at[dstv.at[j]], sem1,
                                     add=True)
                    for j in range(W)
                ]
                for d in descs:
                    d.wait()

            plsc.subcore_barrier()
            pltpu.sync_copy(acc_sh.at[pl.ds(s * RPS, RPS)],
                            deg_hbm.at[c, pl.ds(s * RPS, RPS)])

    return pl.kernel(body, out_type=tuple(out_type), mesh=mesh,
                     scratch_types=tuple(scratch))


def _tc_layer1(x, agg, inv, ws, wn, b):
    def body(x_ref, a_ref, i_ref, ws_ref, wn_ref, b_ref, o_ref):
        hn = a_ref[...] * i_ref[...]
        acc = jnp.dot(x_ref[...], ws_ref[...], preferred_element_type=jnp.float32)
        acc = acc + jnp.dot(hn, wn_ref[...], preferred_element_type=jnp.float32)
        o_ref[...] = jnp.maximum(acc + b_ref[...], 0.0)

    return pl.pallas_call(
        body,
        grid=(N // RB,),
        in_specs=[
            pl.BlockSpec((RB, IN_F), lambda i: (i, 0)),
            pl.BlockSpec((RB, IN_F), lambda i: (i, 0)),
            pl.BlockSpec((RB, 1), lambda i: (i, 0)),
            pl.BlockSpec((IN_F, HID_F), lambda i: (0, 0)),
            pl.BlockSpec((IN_F, HID_F), lambda i: (0, 0)),
            pl.BlockSpec((1, HID_F), lambda i: (0, 0)),
        ],
        out_specs=pl.BlockSpec((RB, HID_F), lambda i: (i, 0)),
        out_shape=jax.ShapeDtypeStruct((N, HID_F), jnp.float32),
    )(x, agg, inv, ws, wn, b)


def _tc_layer2_mlp(h1, agg, inv, ws, wn, b, wm1, bm1, wm2, bm2):
    def body(h_ref, a_ref, i_ref, ws_ref, wn_ref, b_ref,
             wm1_ref, bm1_ref, wm2_ref, bm2_ref, o_ref):
        hn = a_ref[...] * i_ref[...]
        acc = jnp.dot(h_ref[...], ws_ref[...], preferred_element_type=jnp.float32)
        acc = acc + jnp.dot(hn, wn_ref[...], preferred_element_type=jnp.float32)
        h2 = jnp.maximum(acc + b_ref[...], 0.0)
        t = jnp.dot(h2, wm1_ref[...], preferred_element_type=jnp.float32)
        t = jnp.maximum(t + bm1_ref[...], 0.0)
        o = jnp.dot(t, wm2_ref[...], preferred_element_type=jnp.float32)
        o_ref[...] = o + bm2_ref[...]

    return pl.pallas_call(
        body,
        grid=(N // RB,),
        in_specs=[
            pl.BlockSpec((RB, HID_F), lambda i: (i, 0)),
            pl.BlockSpec((RB, HID_F), lambda i: (i, 0)),
            pl.BlockSpec((RB, 1), lambda i: (i, 0)),
            pl.BlockSpec((HID_F, HID_F), lambda i: (0, 0)),
            pl.BlockSpec((HID_F, HID_F), lambda i: (0, 0)),
            pl.BlockSpec((1, HID_F), lambda i: (0, 0)),
            pl.BlockSpec((HID_F, HID_F), lambda i: (0, 0)),
            pl.BlockSpec((1, HID_F), lambda i: (0, 0)),
            pl.BlockSpec((HID_F, OUT_F), lambda i: (0, 0)),
            pl.BlockSpec((1, OUT_F), lambda i: (0, 0)),
        ],
        out_specs=pl.BlockSpec((RB, OUT_F), lambda i: (i, 0)),
        out_shape=jax.ShapeDtypeStruct((N, OUT_F), jnp.float32),
    )(h1, agg, inv, ws, wn, b, wm1, bm1, wm2, bm2)


def kernel(features, edge_index, W_self1, W_neigh1, b1, W_self2, W_neigh2, b2,
           W_mlp1, b_mlp1, W_mlp2, b_mlp2):
    src = edge_index[0].astype(jnp.int32)
    dst = edge_index[1].astype(jnp.int32)
    srcp = jnp.pad(src, (0, EPAD - E))
    dstp = jnp.pad(dst, (0, EPAD - E), constant_values=N)
    di = dstp.reshape(16, S, 128)
    si_base = srcp.reshape(16, S, 128)

    z64 = jnp.zeros((64, 128), jnp.float32)
    ones128 = jnp.ones((128, 128), jnp.float32)

    xcat1 = features.reshape(N, 2, 128).transpose(1, 0, 2).reshape(2 * N, 128)
    si1 = jnp.stack([si_base, si_base + N])[None]
    agg1_c, deg_raw = _make_sc_agg(2, True)(xcat1, si1, di, z64, ones128)
    agg1 = agg1_c[:, :N, :].transpose(1, 0, 2).reshape(N, IN_F)
    deg = deg_raw[0, :N, 0:1] + deg_raw[1, :N, 0:1]
    inv = 1.0 / jnp.maximum(deg, 1.0)

    h1 = _tc_layer1(features, agg1, inv, W_self1, W_neigh1,
                    b1.reshape(1, HID_F))

    xcat2 = h1.reshape(N, 4, 128).transpose(1, 0, 2).reshape(4 * N, 128)
    si2 = jnp.stack([
        jnp.stack([si_base, si_base + N]),
        jnp.stack([si_base + 2 * N, si_base + 3 * N]),
    ])
    (agg2_c,) = _make_sc_agg(4, False)(xcat2, si2, di, z64)
    agg2 = agg2_c[:, :N, :].transpose(1, 0, 2).reshape(N, HID_F)

    out = _tc_layer2_mlp(h1, agg2, inv, W_self2, W_neigh2,
                         b2.reshape(1, HID_F), W_mlp1, b_mlp1.reshape(1, HID_F),
                         W_mlp2, b_mlp2.reshape(1, OUT_F))
    return out
